# Initial kernel scaffold; baseline (speedup 1.0000x reference)
import functools
import jax
import jax.numpy as jnp
from jax import lax
from jax.experimental import pallas as pl
from jax.experimental.pallas import tpu as pltpu
from jax.experimental.pallas import tpu_sc as plsc

N = 49152
C = 128
E = N * 8
K = 4
G = 32

NC = 2
NS = 16
NW = NC * NS
RB = 512
NBLK = N // RB
BPW = NBLK // NW
GE = 64
KE = 1024
GPM = KE // GE


def _lap_body(x_hbm, cols_hbm, vals_hbm, rows_hbm, starts_hbm, y_hbm,
              sbuf, cbuf, vbuf, rbuf, gbuf, acc, gsem):
    wid = lax.axis_index("s") * NC + lax.axis_index("c")
    pltpu.sync_copy(starts_hbm, sbuf)
    zero16 = jnp.zeros((16,), jnp.float32)

    for t in range(BPW):
        b = wid * BPW + t
        r0 = b * RB
        e0 = sbuf[b]
        e1 = sbuf[b + 1]

        def zrow(r, carry):
            for ci in range(8):
                acc[r, pl.ds(ci * 16, 16)] = zero16
            return carry
        lax.fori_loop(0, RB, zrow, 0)

        k0 = e0 // KE
        k1 = (e1 + KE - 1) // KE

        def macro(k, carry):
            pltpu.sync_copy(cols_hbm.at[pl.ds(k * GPM, GPM)], cbuf)
            pltpu.sync_copy(vals_hbm.at[pl.ds(k * GPM, GPM)], vbuf)
            pltpu.sync_copy(rows_hbm.at[pl.ds(k * GPM, GPM)], rbuf)

            def gath(s, carry2):
                pltpu.async_copy(x_hbm.at[cbuf.at[s]], gbuf, gsem).wait()

                def edge(j, carry3):
                    e = k * KE + s * GE + j
                    valid = (e >= e0) & (e < e1)
                    rloc = jnp.where(valid, rbuf[s, j] - r0, 0)
                    vv = jnp.where(valid, vbuf[s, j], 0.0)
                    for ci in range(8):
                        g = gbuf[j, pl.ds(ci * 16, 16)]
                        plsc.addupdate(acc.at[rloc, pl.ds(ci * 16, 16)], g * vv)
                    return carry3
                lax.fori_loop(0, GE, edge, 0)
                return carry2
            lax.fori_loop(0, GPM, gath, 0)
            return carry
        lax.fori_loop(k0, k1, macro, 0)

        pltpu.sync_copy(acc, y_hbm.at[pl.ds(r0, RB)])


def _make_lap():
    mesh = plsc.VectorSubcoreMesh(core_axis_name="c", subcore_axis_name="s")
    return pl.kernel(
        _lap_body,
        out_type=jax.ShapeDtypeStruct((N, C), jnp.float32),
        mesh=mesh,
        scratch_types=[
            pltpu.VMEM((128,), jnp.int32),
            pltpu.VMEM((GPM, GE), jnp.int32),
            pltpu.VMEM((GPM, GE), jnp.float32),
            pltpu.VMEM((GPM, GE), jnp.int32),
            pltpu.VMEM((GE, C), jnp.float32),
            pltpu.VMEM((RB, C), jnp.float32),
            pltpu.SemaphoreType.DMA,
        ],
    )


_lap_call = _make_lap()


def _lap(x, cols2, vals2, rows2, starts):
    return _lap_call(x, cols2, vals2, rows2, starts)


RB_GN = 2048


def _gn_stats_body(x_ref, o_ref):
    i = pl.program_id(0)

    @pl.when(i == 0)
    def _():
        o_ref[...] = jnp.zeros_like(o_ref)

    xb = x_ref[...]
    o_ref[0:1, :] += jnp.sum(xb, axis=0, keepdims=True)
    o_ref[1:2, :] += jnp.sum(xb * xb, axis=0, keepdims=True)


def _gn_apply_body(x_ref, st_ref, g_ref, b_ref, o_ref):
    ri = lax.broadcasted_iota(jnp.int32, (C, C), 0)
    ci = lax.broadcasted_iota(jnp.int32, (C, C), 1)
    A = jnp.where((ri // (C // G)) == (ci // (C // G)), 1.0, 0.0)
    chs = st_ref[0:1, :]
    chq = st_ref[1:2, :]
    gs = jnp.dot(chs, A, preferred_element_type=jnp.float32)
    gq = jnp.dot(chq, A, preferred_element_type=jnp.float32)
    denom = float(N * (C // G))
    mean = gs / denom
    var = gq / denom - mean * mean
    scale = g_ref[0:1, :] * lax.rsqrt(var + 1e-5)
    shift = b_ref[0:1, :] - mean * scale
    t = x_ref[...] * scale + shift
    o_ref[...] = t * (1.0 / (1.0 + jnp.exp(-t)))


def _gn_silu(x, gamma, beta):
    stats = pl.pallas_call(
        _gn_stats_body,
        grid=(N // RB_GN,),
        in_specs=[pl.BlockSpec((RB_GN, C), lambda i: (i, 0))],
        out_specs=pl.BlockSpec((8, C), lambda i: (0, 0)),
        out_shape=jax.ShapeDtypeStruct((8, C), jnp.float32),
    )(x)
    return pl.pallas_call(
        _gn_apply_body,
        grid=(N // RB_GN,),
        in_specs=[
            pl.BlockSpec((RB_GN, C), lambda i: (i, 0)),
            pl.BlockSpec((8, C), lambda i: (0, 0)),
            pl.BlockSpec((8, C), lambda i: (0, 0)),
            pl.BlockSpec((8, C), lambda i: (0, 0)),
        ],
        out_specs=pl.BlockSpec((RB_GN, C), lambda i: (i, 0)),
        out_shape=jax.ShapeDtypeStruct((N, C), jnp.float32),
    )(x, stats, gamma, beta)


RB_MM = 2048


def _mm_body(has_res, *refs):
    if has_res:
        t0, t1, t2, t3, w_ref, res_ref, o_ref = refs
    else:
        t0, t1, t2, t3, w_ref, o_ref = refs
    acc = jnp.dot(t0[...], w_ref[0:C, :], preferred_element_type=jnp.float32)
    acc += jnp.dot(t1[...], w_ref[C:2 * C, :], preferred_element_type=jnp.float32)
    acc += jnp.dot(t2[...], w_ref[2 * C:3 * C, :], preferred_element_type=jnp.float32)
    acc += jnp.dot(t3[...], w_ref[3 * C:4 * C, :], preferred_element_type=jnp.float32)
    if has_res:
        acc += res_ref[...]
    o_ref[...] = acc


def _mm(t0, t1, t2, t3, w, res=None):
    has_res = res is not None
    blk = pl.BlockSpec((RB_MM, C), lambda i: (i, 0))
    in_specs = [blk, blk, blk, blk, pl.BlockSpec((K * C, C), lambda i: (0, 0))]
    args = [t0, t1, t2, t3, w]
    if has_res:
        in_specs.append(blk)
        args.append(res)
    return pl.pallas_call(
        functools.partial(_mm_body, has_res),
        grid=(N // RB_MM,),
        in_specs=in_specs,
        out_specs=blk,
        out_shape=jax.ShapeDtypeStruct((N, C), jnp.float32),
    )(*args)


def _cheb(h, cols2, vals2, rows2, starts, w, res=None):
    u1 = _lap(h, cols2, vals2, rows2, starts)
    u2 = _lap(u1, cols2, vals2, rows2, starts)
    t2 = 2.0 * u2 - h
    u3 = _lap(t2, cols2, vals2, rows2, starts)
    t3 = 2.0 * u3 - u1
    return _mm(h, u1, t2, t3, w, res=res)


def kernel(x, lap_rows, lap_cols, lap_vals, w1, w2, gamma1, beta1, gamma2, beta2):
    x2 = x[0]
    boundaries = jnp.arange(0, N + RB, RB, dtype=jnp.int32)
    starts = jnp.searchsorted(lap_rows, boundaries).astype(jnp.int32)
    starts = jnp.pad(starts, (0, 128 - starts.shape[0]))
    cols2 = lap_cols.reshape(E // GE, GE)
    vals2 = lap_vals.reshape(E // GE, GE)
    rows2 = lap_rows.reshape(E // GE, GE)
    g1 = jnp.broadcast_to(gamma1[None, :], (8, C))
    b1 = jnp.broadcast_to(beta1[None, :], (8, C))
    g2 = jnp.broadcast_to(gamma2[None, :], (8, C))
    b2 = jnp.broadcast_to(beta2[None, :], (8, C))

    h = _gn_silu(x2, g1, b1)
    y1 = _cheb(h, cols2, vals2, rows2, starts, w1)
    h2 = _gn_silu(y1, g2, b2)
    out = _cheb(h2, cols2, vals2, rows2, starts, w2, res=x2)
    return out[None]


# speedup vs baseline: 2.9980x; 2.9980x over previous
import functools
import jax
import jax.numpy as jnp
from jax import lax
from jax.experimental import pallas as pl
from jax.experimental.pallas import tpu as pltpu
from jax.experimental.pallas import tpu_sc as plsc

N = 49152
C = 128
E = N * 8
K = 4
G = 32

NC = 2
NS = 16
NW = NC * NS
RB = 512
NBLK = N // RB
BPW = NBLK // NW
GE = 64
KE = 1024
GPM = KE // GE


NBUF = 4


def _lap_body(x_hbm, zeros_hbm, cols_hbm, vals_hbm, rows_hbm, starts_hbm,
              y_hbm, sbuf, cbuf, vbuf, rbuf, gb0, gb1, gb2, gb3, acc,
              sm0, sm1, sm2, sm3):
    gbufs = [gb0, gb1, gb2, gb3]
    sems = [sm0, sm1, sm2, sm3]
    wid = lax.axis_index("s") * NC + lax.axis_index("c")
    pltpu.sync_copy(starts_hbm, sbuf)

    for t in range(BPW):
        b = wid * BPW + t
        r0 = b * RB
        ev = sbuf[b, pl.ds(0, 16)]
        e0 = ev[0]
        e1 = ev[1]

        pltpu.sync_copy(zeros_hbm, acc)

        k0 = e0 // KE
        k1 = (e1 + KE - 1) // KE

        def compute(s, gb, k):
            def edge16(j16, carry3):
                rv = rbuf[s, pl.ds(j16 * 16, 16)]
                vv16 = vbuf[s, pl.ds(j16 * 16, 16)]
                base_e = k * KE + s * GE + j16 * 16
                for jj in range(16):
                    e = base_e + jj
                    valid = (e >= e0) & (e < e1)
                    rloc = jnp.where(valid, rv[jj] - r0, 0)
                    vvs = jnp.where(valid, vv16[jj], 0.0)
                    for ci in range(8):
                        g = gb[j16 * 16 + jj, pl.ds(ci * 16, 16)]
                        plsc.addupdate(acc.at[rloc, pl.ds(ci * 16, 16)],
                                       g * vvs)
                return carry3
            lax.fori_loop(0, GE // 16, edge16, 0)

        def macro(k, carry):
            cps = [
                pltpu.async_copy(cols_hbm.at[pl.ds(k * GPM, GPM)], cbuf,
                                 sems[0]),
                pltpu.async_copy(vals_hbm.at[pl.ds(k * GPM, GPM)], vbuf,
                                 sems[1]),
                pltpu.async_copy(rows_hbm.at[pl.ds(k * GPM, GPM)], rbuf,
                                 sems[2]),
            ]
            for cp in cps:
                cp.wait()

            def quad(s4, carry2):
                s0 = s4 * NBUF
                gcps = [
                    pltpu.async_copy(x_hbm.at[cbuf.at[s0 + q]], gbufs[q],
                                     sems[q])
                    for q in range(NBUF)
                ]
                for q in range(NBUF):
                    gcps[q].wait()
                    compute(s0 + q, gbufs[q], k)
                return carry2
            lax.fori_loop(0, GPM // NBUF, quad, 0)
            return carry
        lax.fori_loop(k0, k1, macro, 0)

        pltpu.sync_copy(acc, y_hbm.at[pl.ds(r0, RB)])


@functools.cache
def _make_lap():
    mesh = plsc.VectorSubcoreMesh(core_axis_name="c", subcore_axis_name="s")
    return pl.kernel(
        _lap_body,
        out_type=jax.ShapeDtypeStruct((N, C), jnp.float32),
        mesh=mesh,
        scratch_types=[
            pltpu.VMEM((NBLK, 16), jnp.int32),
            pltpu.VMEM((GPM, GE), jnp.int32),
            pltpu.VMEM((GPM, GE), jnp.float32),
            pltpu.VMEM((GPM, GE), jnp.int32),
            pltpu.VMEM((GE, C), jnp.float32),
            pltpu.VMEM((GE, C), jnp.float32),
            pltpu.VMEM((GE, C), jnp.float32),
            pltpu.VMEM((GE, C), jnp.float32),
            pltpu.VMEM((RB, C), jnp.float32),
            pltpu.SemaphoreType.DMA,
            pltpu.SemaphoreType.DMA,
            pltpu.SemaphoreType.DMA,
            pltpu.SemaphoreType.DMA,
        ],
    )


def _lap(x, zeros_rb, cols2, vals2, rows2, starts):
    return _make_lap()(x, zeros_rb, cols2, vals2, rows2, starts)


RB_GN = 2048


def _gn_stats_body(x_ref, o_ref):
    i = pl.program_id(0)

    @pl.when(i == 0)
    def _():
        o_ref[...] = jnp.zeros_like(o_ref)

    xb = x_ref[...]
    o_ref[0:1, :] += jnp.sum(xb, axis=0, keepdims=True)
    o_ref[1:2, :] += jnp.sum(xb * xb, axis=0, keepdims=True)


def _gn_apply_body(x_ref, st_ref, g_ref, b_ref, o_ref):
    ri = lax.broadcasted_iota(jnp.int32, (C, C), 0)
    ci = lax.broadcasted_iota(jnp.int32, (C, C), 1)
    A = jnp.where((ri // (C // G)) == (ci // (C // G)), 1.0, 0.0)
    chs = st_ref[0:1, :]
    chq = st_ref[1:2, :]
    gs = jnp.dot(chs, A, preferred_element_type=jnp.float32)
    gq = jnp.dot(chq, A, preferred_element_type=jnp.float32)
    denom = float(N * (C // G))
    mean = gs / denom
    var = gq / denom - mean * mean
    scale = g_ref[0:1, :] * lax.rsqrt(var + 1e-5)
    shift = b_ref[0:1, :] - mean * scale
    t = x_ref[...] * scale + shift
    o_ref[...] = t * (1.0 / (1.0 + jnp.exp(-t)))


def _gn_silu(x, gamma, beta):
    stats = pl.pallas_call(
        _gn_stats_body,
        grid=(N // RB_GN,),
        in_specs=[pl.BlockSpec((RB_GN, C), lambda i: (i, 0))],
        out_specs=pl.BlockSpec((8, C), lambda i: (0, 0)),
        out_shape=jax.ShapeDtypeStruct((8, C), jnp.float32),
    )(x)
    return pl.pallas_call(
        _gn_apply_body,
        grid=(N // RB_GN,),
        in_specs=[
            pl.BlockSpec((RB_GN, C), lambda i: (i, 0)),
            pl.BlockSpec((8, C), lambda i: (0, 0)),
            pl.BlockSpec((8, C), lambda i: (0, 0)),
            pl.BlockSpec((8, C), lambda i: (0, 0)),
        ],
        out_specs=pl.BlockSpec((RB_GN, C), lambda i: (i, 0)),
        out_shape=jax.ShapeDtypeStruct((N, C), jnp.float32),
    )(x, stats, gamma, beta)


RB_MM = 2048


def _mm_body(has_res, *refs):
    if has_res:
        t0, t1, t2, t3, w_ref, res_ref, o_ref = refs
    else:
        t0, t1, t2, t3, w_ref, o_ref = refs
    acc = jnp.dot(t0[...], w_ref[0:C, :], preferred_element_type=jnp.float32)
    acc += jnp.dot(t1[...], w_ref[C:2 * C, :], preferred_element_type=jnp.float32)
    acc += jnp.dot(t2[...], w_ref[2 * C:3 * C, :], preferred_element_type=jnp.float32)
    acc += jnp.dot(t3[...], w_ref[3 * C:4 * C, :], preferred_element_type=jnp.float32)
    if has_res:
        acc += res_ref[...]
    o_ref[...] = acc


def _mm(t0, t1, t2, t3, w, res=None):
    has_res = res is not None
    blk = pl.BlockSpec((RB_MM, C), lambda i: (i, 0))
    in_specs = [blk, blk, blk, blk, pl.BlockSpec((K * C, C), lambda i: (0, 0))]
    args = [t0, t1, t2, t3, w]
    if has_res:
        in_specs.append(blk)
        args.append(res)
    return pl.pallas_call(
        functools.partial(_mm_body, has_res),
        grid=(N // RB_MM,),
        in_specs=in_specs,
        out_specs=blk,
        out_shape=jax.ShapeDtypeStruct((N, C), jnp.float32),
    )(*args)


def _cheb(h, zeros_rb, cols2, vals2, rows2, starts, w, res=None):
    u1 = _lap(h, zeros_rb, cols2, vals2, rows2, starts)
    u2 = _lap(u1, zeros_rb, cols2, vals2, rows2, starts)
    t2 = 2.0 * u2 - h
    u3 = _lap(t2, zeros_rb, cols2, vals2, rows2, starts)
    t3 = 2.0 * u3 - u1
    return _mm(h, u1, t2, t3, w, res=res)


def kernel(x, lap_rows, lap_cols, lap_vals, w1, w2, gamma1, beta1, gamma2, beta2):
    x2 = x[0]
    boundaries = jnp.arange(0, N + RB, RB, dtype=jnp.int32)
    se = jnp.searchsorted(lap_rows, boundaries).astype(jnp.int32)
    starts = jnp.zeros((NBLK, 16), jnp.int32)
    starts = starts.at[:, 0].set(se[:-1]).at[:, 1].set(se[1:])
    cols2 = lap_cols.reshape(E // GE, GE)
    vals2 = lap_vals.reshape(E // GE, GE)
    rows2 = lap_rows.reshape(E // GE, GE)
    g1 = jnp.broadcast_to(gamma1[None, :], (8, C))
    b1 = jnp.broadcast_to(beta1[None, :], (8, C))
    g2 = jnp.broadcast_to(gamma2[None, :], (8, C))
    b2 = jnp.broadcast_to(beta2[None, :], (8, C))

    zeros_rb = jnp.zeros((RB, C), jnp.float32)

    h = _gn_silu(x2, g1, b1)
    y1 = _cheb(h, zeros_rb, cols2, vals2, rows2, starts, w1)
    h2 = _gn_silu(y1, g2, b2)
    out = _cheb(h2, zeros_rb, cols2, vals2, rows2, starts, w2, res=x2)
    return out[None]


# speedup vs baseline: 6.3830x; 2.1291x over previous
import functools
import jax
import jax.numpy as jnp
from jax import lax
from jax.experimental import pallas as pl
from jax.experimental.pallas import tpu as pltpu
from jax.experimental.pallas import tpu_sc as plsc

N = 49152
C = 128
E = N * 8
K = 4
G = 32

NC = 2
NS = 16
NW = NC * NS
RB = 512
NBLK = N // RB
BPW = NBLK // NW
GE = 64
KE = 1024
GPM = KE // GE


NBUF = 4


def _lap_body(x_hbm, zeros_hbm, cols_hbm, vals_hbm, rows_hbm, starts_hbm,
              y_hbm, sbuf, cbuf, vbuf, rbuf, gb0, gb1, gb2, gb3, acc,
              sm0, sm1, sm2, sm3):
    gbufs = [gb0, gb1, gb2, gb3]
    sems = [sm0, sm1, sm2, sm3]
    wid = lax.axis_index("s") * NC + lax.axis_index("c")
    pltpu.sync_copy(starts_hbm, sbuf)

    def block_body(t, carry0):
        b = wid * BPW + t
        r0 = b * RB
        ev = sbuf[b, pl.ds(0, 16)]
        e0 = ev[0]
        e1 = ev[1]

        pltpu.sync_copy(zeros_hbm, acc)

        k0 = e0 // KE
        k1 = (e1 + KE - 1) // KE

        def compute(s, gb, k, masked):
            @plsc.parallel_loop(0, GE, 1, unroll=4)
            def edge(j):
                jv = jnp.full((16,), s * GE + j, jnp.int32)
                vvec = plsc.load_gather(vbuf, [jv])
                rvec = plsc.load_gather(rbuf, [jv])
                if masked:
                    e = k * KE + s * GE + j
                    valid = (e >= e0) & (e < e1)
                    rloc = jnp.where(valid, rvec[0] - r0, 0)
                    vv = jnp.where(valid, vvec, 0.0)
                else:
                    rloc = rvec[0] - r0
                    vv = vvec
                for ci in range(8):
                    g = gb[j, pl.ds(ci * 16, 16)]
                    plsc.addupdate(acc.at[rloc, pl.ds(ci * 16, 16)], g * vv)

        def macro(k, carry):
            cps = [
                pltpu.async_copy(cols_hbm.at[pl.ds(k * GPM, GPM)], cbuf,
                                 sems[0]),
                pltpu.async_copy(vals_hbm.at[pl.ds(k * KE, KE)], vbuf,
                                 sems[1]),
                pltpu.async_copy(rows_hbm.at[pl.ds(k * KE, KE)], rbuf,
                                 sems[2]),
            ]
            for cp in cps:
                cp.wait()

            def quads(masked):
                def quad(s4, carry2):
                    s0 = s4 * NBUF
                    gcps = [
                        pltpu.async_copy(x_hbm.at[cbuf.at[s0 + q]], gbufs[q],
                                         sems[q])
                        for q in range(NBUF)
                    ]
                    for q in range(NBUF):
                        gcps[q].wait()
                        compute(s0 + q, gbufs[q], k, masked)
                    return carry2
                lax.fori_loop(0, GPM // NBUF, quad, 0)
                return 0

            boundary = (k == k0) | (k == k1 - 1)
            lax.cond(boundary, lambda: quads(True), lambda: quads(False))
            return carry
        lax.fori_loop(k0, k1, macro, 0)

        pltpu.sync_copy(acc, y_hbm.at[pl.ds(r0, RB)])
        return carry0

    lax.fori_loop(0, BPW, block_body, 0)


@functools.cache
def _make_lap():
    mesh = plsc.VectorSubcoreMesh(core_axis_name="c", subcore_axis_name="s")
    return pl.kernel(
        _lap_body,
        out_type=jax.ShapeDtypeStruct((N, C), jnp.float32),
        mesh=mesh,
        compiler_params=pltpu.CompilerParams(needs_layout_passes=False),
        scratch_types=[
            pltpu.VMEM((NBLK, 16), jnp.int32),
            pltpu.VMEM((GPM, GE), jnp.int32),
            pltpu.VMEM((KE,), jnp.float32),
            pltpu.VMEM((KE,), jnp.int32),
            pltpu.VMEM((GE, C), jnp.float32),
            pltpu.VMEM((GE, C), jnp.float32),
            pltpu.VMEM((GE, C), jnp.float32),
            pltpu.VMEM((GE, C), jnp.float32),
            pltpu.VMEM((RB, C), jnp.float32),
            pltpu.SemaphoreType.DMA,
            pltpu.SemaphoreType.DMA,
            pltpu.SemaphoreType.DMA,
            pltpu.SemaphoreType.DMA,
        ],
    )


def _lap(x, zeros_rb, cols2, vals2, rows2, starts):
    return _make_lap()(x, zeros_rb, cols2, vals2, rows2, starts)


RB_GN = 2048


def _gn_stats_body(x_ref, o_ref):
    i = pl.program_id(0)

    @pl.when(i == 0)
    def _():
        o_ref[...] = jnp.zeros_like(o_ref)

    xb = x_ref[...]
    o_ref[0:1, :] += jnp.sum(xb, axis=0, keepdims=True)
    o_ref[1:2, :] += jnp.sum(xb * xb, axis=0, keepdims=True)


def _gn_apply_body(x_ref, st_ref, g_ref, b_ref, o_ref):
    ri = lax.broadcasted_iota(jnp.int32, (C, C), 0)
    ci = lax.broadcasted_iota(jnp.int32, (C, C), 1)
    A = jnp.where((ri // (C // G)) == (ci // (C // G)), 1.0, 0.0)
    chs = st_ref[0:1, :]
    chq = st_ref[1:2, :]
    gs = jnp.dot(chs, A, preferred_element_type=jnp.float32)
    gq = jnp.dot(chq, A, preferred_element_type=jnp.float32)
    denom = float(N * (C // G))
    mean = gs / denom
    var = gq / denom - mean * mean
    scale = g_ref[0:1, :] * lax.rsqrt(var + 1e-5)
    shift = b_ref[0:1, :] - mean * scale
    t = x_ref[...] * scale + shift
    o_ref[...] = t * (1.0 / (1.0 + jnp.exp(-t)))


def _gn_silu(x, gamma, beta):
    stats = pl.pallas_call(
        _gn_stats_body,
        grid=(N // RB_GN,),
        in_specs=[pl.BlockSpec((RB_GN, C), lambda i: (i, 0))],
        out_specs=pl.BlockSpec((8, C), lambda i: (0, 0)),
        out_shape=jax.ShapeDtypeStruct((8, C), jnp.float32),
    )(x)
    return pl.pallas_call(
        _gn_apply_body,
        grid=(N // RB_GN,),
        in_specs=[
            pl.BlockSpec((RB_GN, C), lambda i: (i, 0)),
            pl.BlockSpec((8, C), lambda i: (0, 0)),
            pl.BlockSpec((8, C), lambda i: (0, 0)),
            pl.BlockSpec((8, C), lambda i: (0, 0)),
        ],
        out_specs=pl.BlockSpec((RB_GN, C), lambda i: (i, 0)),
        out_shape=jax.ShapeDtypeStruct((N, C), jnp.float32),
    )(x, stats, gamma, beta)


RB_MM = 2048


def _mm_body(has_res, *refs):
    if has_res:
        t0, t1, t2, t3, w_ref, res_ref, o_ref = refs
    else:
        t0, t1, t2, t3, w_ref, o_ref = refs
    acc = jnp.dot(t0[...], w_ref[0:C, :], preferred_element_type=jnp.float32)
    acc += jnp.dot(t1[...], w_ref[C:2 * C, :], preferred_element_type=jnp.float32)
    acc += jnp.dot(t2[...], w_ref[2 * C:3 * C, :], preferred_element_type=jnp.float32)
    acc += jnp.dot(t3[...], w_ref[3 * C:4 * C, :], preferred_element_type=jnp.float32)
    if has_res:
        acc += res_ref[...]
    o_ref[...] = acc


def _mm(t0, t1, t2, t3, w, res=None):
    has_res = res is not None
    blk = pl.BlockSpec((RB_MM, C), lambda i: (i, 0))
    in_specs = [blk, blk, blk, blk, pl.BlockSpec((K * C, C), lambda i: (0, 0))]
    args = [t0, t1, t2, t3, w]
    if has_res:
        in_specs.append(blk)
        args.append(res)
    return pl.pallas_call(
        functools.partial(_mm_body, has_res),
        grid=(N // RB_MM,),
        in_specs=in_specs,
        out_specs=blk,
        out_shape=jax.ShapeDtypeStruct((N, C), jnp.float32),
    )(*args)


def _cheb(h, zeros_rb, cols2, vals2, rows2, starts, w, res=None):
    u1 = _lap(h, zeros_rb, cols2, vals2, rows2, starts)
    u2 = _lap(u1, zeros_rb, cols2, vals2, rows2, starts)
    t2 = 2.0 * u2 - h
    u3 = _lap(t2, zeros_rb, cols2, vals2, rows2, starts)
    t3 = 2.0 * u3 - u1
    return _mm(h, u1, t2, t3, w, res=res)


def kernel(x, lap_rows, lap_cols, lap_vals, w1, w2, gamma1, beta1, gamma2, beta2):
    x2 = x[0]
    boundaries = jnp.arange(0, N + RB, RB, dtype=jnp.int32)
    se = jnp.searchsorted(lap_rows, boundaries).astype(jnp.int32)
    starts = jnp.zeros((NBLK, 16), jnp.int32)
    starts = starts.at[:, 0].set(se[:-1]).at[:, 1].set(se[1:])
    cols2 = lap_cols.reshape(E // GE, GE)
    vals2 = lap_vals
    rows2 = lap_rows
    g1 = jnp.broadcast_to(gamma1[None, :], (8, C))
    b1 = jnp.broadcast_to(beta1[None, :], (8, C))
    g2 = jnp.broadcast_to(gamma2[None, :], (8, C))
    b2 = jnp.broadcast_to(beta2[None, :], (8, C))

    zeros_rb = jnp.zeros((RB, C), jnp.float32)

    h = _gn_silu(x2, g1, b1)
    y1 = _cheb(h, zeros_rb, cols2, vals2, rows2, starts, w1)
    h2 = _gn_silu(y1, g2, b2)
    out = _cheb(h2, zeros_rb, cols2, vals2, rows2, starts, w2, res=x2)
    return out[None]
